# baseline (speedup 1.0000x reference)
import jax
import jax.numpy as jnp
from jax import lax
from jax.experimental import pallas as pl
from jax.experimental.pallas import tpu as pltpu
from jax.experimental.pallas import tpu_sc as plsc

_NC = 2
_NS = 16
_NW = _NC * _NS
_L = 16

_N_CELLS = 16384
_D = 64
_VOCAB = 100000
_RPW = _D // _NW
_ICHUNK = 2048
_NICHUNK = _N_CELLS // _ICHUNK


def _body(z_hbm, idx_hbm, table_hbm, out_hbm, row_v, acc_v, idx_a, idx_b,
          sems):
    wid = lax.axis_index("s") * _NC + lax.axis_index("c")

    idx_bufs = (idx_a, idx_b)
    for r in range(_RPW):
        j = wid * _RPW + r
        rowcp = pltpu.async_copy(
            table_hbm.at[pl.ds(j, 1), pl.ds(0, 49152)],
            row_v.at[:, pl.ds(0, 49152)], sems[2])
        rowcp2 = pltpu.async_copy(
            table_hbm.at[pl.ds(j, 1), pl.ds(49152, 50816)],
            row_v.at[:, pl.ds(49152, 50816)], sems[3])
        zcp = pltpu.async_copy(z_hbm.at[pl.ds(j, 1), :], acc_v, sems[4])
        icp = pltpu.async_copy(idx_hbm.at[pl.ds(0, _ICHUNK)], idx_a, sems[0])
        rowcp.wait()
        rowcp2.wait()
        zcp.wait()

        for ch in range(_NICHUNK):
            if ch + 1 < _NICHUNK:
                nxt = pltpu.async_copy(
                    idx_hbm.at[pl.ds((ch + 1) * _ICHUNK, _ICHUNK)],
                    idx_bufs[(ch + 1) % 2], sems[(ch + 1) % 2])
            icp.wait()
            buf = idx_bufs[ch % 2]
            base = ch * _ICHUNK

            def grp(g, carry, buf=buf, base=base):
                k = g * _L
                iv = jnp.minimum(buf[pl.ds(k, _L)], 99967)
                gathered = plsc.load_gather(
                    row_v, [jnp.zeros((_L,), jnp.int32), iv])
                plsc.addupdate(acc_v.at[0, pl.ds(base + k, _L)], gathered)
                return carry

            lax.fori_loop(0, _ICHUNK // _L, grp, 0, unroll=8)
            if ch + 1 < _NICHUNK:
                icp = nxt

        pltpu.sync_copy(acc_v, out_hbm.at[pl.ds(j, 1), :])


@jax.jit
def kernel(z_bio, batch_ids, emb_weight):
    idx = batch_ids if batch_ids.dtype == jnp.int32 else batch_ids.astype(jnp.int32)
    zt = z_bio.T
    tt = emb_weight.T
    mesh = plsc.VectorSubcoreMesh(
        core_axis_name="c", subcore_axis_name="s",
        num_cores=_NC, num_subcores=_NS,
    )
    f = pl.kernel(
        _body,
        out_type=jax.ShapeDtypeStruct((_D, _N_CELLS), jnp.float32),
        mesh=mesh,
        scratch_types=[
            pltpu.VMEM((1, _VOCAB), jnp.float32),
            pltpu.VMEM((1, _N_CELLS), jnp.float32),
            pltpu.VMEM((_ICHUNK,), jnp.int32),
            pltpu.VMEM((_ICHUNK,), jnp.int32),
            [pltpu.SemaphoreType.DMA] * 5,
        ],
        compiler_params=pltpu.CompilerParams(
            use_tc_tiling_on_sc=True,
            skip_device_barrier=True,
            needs_layout_passes=False,
        ),
    )
    return f(zt, idx, tt).T


# speedup vs baseline: 1.0710x; 1.0710x over previous
import jax
import jax.numpy as jnp
from jax import lax
from jax.experimental import pallas as pl
from jax.experimental.pallas import tpu as pltpu
from jax.experimental.pallas import tpu_sc as plsc

_NC = 2
_NS = 16
_NW = _NC * _NS
_L = 16

_N_CELLS = 16384
_D = 64
_VOCAB = 100000
_RPW = _D // _NW
_ICHUNK = 4096
_NICHUNK = _N_CELLS // _ICHUNK


def _body(z_hbm, idx_hbm, table_hbm, out_hbm, row_v, acc_v, idx_a, idx_b,
          sems):
    wid = lax.axis_index("s") * _NC + lax.axis_index("c")

    idx_bufs = (idx_a, idx_b)
    for r in range(_RPW):
        j = wid * _RPW + r
        rowcp = pltpu.async_copy(table_hbm.at[pl.ds(j, 1), :], row_v, sems[2])
        zcp = pltpu.async_copy(z_hbm.at[pl.ds(j, 1), :], acc_v, sems[3])
        icp = pltpu.async_copy(idx_hbm.at[pl.ds(0, _ICHUNK)], idx_a, sems[0])
        rowcp.wait()
        zcp.wait()

        for ch in range(_NICHUNK):
            if ch + 1 < _NICHUNK:
                nxt = pltpu.async_copy(
                    idx_hbm.at[pl.ds((ch + 1) * _ICHUNK, _ICHUNK)],
                    idx_bufs[(ch + 1) % 2], sems[(ch + 1) % 2])
            icp.wait()
            buf = idx_bufs[ch % 2]
            base = ch * _ICHUNK

            def grp(g, carry, buf=buf, base=base):
                k = g * _L
                iv = buf[pl.ds(k, _L)]
                gathered = plsc.load_gather(
                    row_v, [jnp.zeros((_L,), jnp.int32), iv])
                plsc.addupdate(acc_v.at[0, pl.ds(base + k, _L)], gathered)
                return carry

            lax.fori_loop(0, _ICHUNK // _L, grp, 0, unroll=8)
            if ch + 1 < _NICHUNK:
                icp = nxt

        pltpu.sync_copy(acc_v, out_hbm.at[pl.ds(j, 1), :])


@jax.jit
def kernel(z_bio, batch_ids, emb_weight):
    idx = batch_ids if batch_ids.dtype == jnp.int32 else batch_ids.astype(jnp.int32)
    zt = z_bio.T
    tt = emb_weight.T
    mesh = plsc.VectorSubcoreMesh(
        core_axis_name="c", subcore_axis_name="s",
        num_cores=_NC, num_subcores=_NS,
    )
    f = pl.kernel(
        _body,
        out_type=jax.ShapeDtypeStruct((_D, _N_CELLS), jnp.float32),
        mesh=mesh,
        scratch_types=[
            pltpu.VMEM((1, _VOCAB), jnp.float32),
            pltpu.VMEM((1, _N_CELLS), jnp.float32),
            pltpu.VMEM((_ICHUNK,), jnp.int32),
            pltpu.VMEM((_ICHUNK,), jnp.int32),
            [pltpu.SemaphoreType.DMA] * 4,
        ],
        compiler_params=pltpu.CompilerParams(
            use_tc_tiling_on_sc=True,
            skip_device_barrier=True,
            needs_layout_passes=False,
        ),
    )
    return f(zt, idx, tt).T


# speedup vs baseline: 1.0823x; 1.0105x over previous
import jax
import jax.numpy as jnp
from jax import lax
from jax.experimental import pallas as pl
from jax.experimental.pallas import tpu as pltpu
from jax.experimental.pallas import tpu_sc as plsc

_NC = 2
_NS = 16
_NW = _NC * _NS
_L = 16

_N_CELLS = 16384
_D = 64
_VOCAB = 100000
_RPW = _D // _NW
_ICHUNK = 4096
_NICHUNK = _N_CELLS // _ICHUNK


def _body(z_hbm, idx_hbm, table_hbm, out_hbm, row_v, acc_v, idx_a, idx_b,
          sems):
    wid = lax.axis_index("s") * _NC + lax.axis_index("c")

    idx_bufs = (idx_a, idx_b)
    outcp = None
    for r in range(_RPW):
        j = wid * _RPW + r
        rowcp = pltpu.async_copy(table_hbm.at[pl.ds(j, 1), :], row_v, sems[2])
        if outcp is not None:
            outcp.wait()
        zcp = pltpu.async_copy(z_hbm.at[pl.ds(j, 1), :], acc_v, sems[3])
        icp = pltpu.async_copy(idx_hbm.at[pl.ds(0, _ICHUNK)], idx_a, sems[0])
        rowcp.wait()
        zcp.wait()

        for ch in range(_NICHUNK):
            if ch + 1 < _NICHUNK:
                nxt = pltpu.async_copy(
                    idx_hbm.at[pl.ds((ch + 1) * _ICHUNK, _ICHUNK)],
                    idx_bufs[(ch + 1) % 2], sems[(ch + 1) % 2])
            icp.wait()
            buf = idx_bufs[ch % 2]
            base = ch * _ICHUNK

            def grp(g, carry, buf=buf, base=base):
                k = g * _L
                iv = buf[pl.ds(k, _L)]
                gathered = plsc.load_gather(
                    row_v, [jnp.zeros((_L,), jnp.int32), iv])
                plsc.addupdate(acc_v.at[0, pl.ds(base + k, _L)], gathered)
                return carry

            lax.fori_loop(0, _ICHUNK // _L, grp, 0, unroll=8)
            if ch + 1 < _NICHUNK:
                icp = nxt

        outcp = pltpu.async_copy(acc_v, out_hbm.at[pl.ds(j, 1), :], sems[3])
    outcp.wait()


@jax.jit
def kernel(z_bio, batch_ids, emb_weight):
    idx = batch_ids if batch_ids.dtype == jnp.int32 else batch_ids.astype(jnp.int32)
    zt = z_bio.T
    tt = emb_weight.T
    mesh = plsc.VectorSubcoreMesh(
        core_axis_name="c", subcore_axis_name="s",
        num_cores=_NC, num_subcores=_NS,
    )
    f = pl.kernel(
        _body,
        out_type=jax.ShapeDtypeStruct((_D, _N_CELLS), jnp.float32),
        mesh=mesh,
        scratch_types=[
            pltpu.VMEM((1, _VOCAB), jnp.float32),
            pltpu.VMEM((1, _N_CELLS), jnp.float32),
            pltpu.VMEM((_ICHUNK,), jnp.int32),
            pltpu.VMEM((_ICHUNK,), jnp.int32),
            [pltpu.SemaphoreType.DMA] * 4,
        ],
        compiler_params=pltpu.CompilerParams(
            use_tc_tiling_on_sc=True,
            skip_device_barrier=True,
            needs_layout_passes=False,
        ),
    )
    return f(zt, idx, tt).T
